# baseline (speedup 1.0000x reference)
import functools

import jax
import jax.numpy as jnp
from jax import lax
from jax.experimental import pallas as pl
from jax.experimental.pallas import tpu as pltpu
from jax.experimental.pallas import tpu_sc as plsc

N = 10000
E = 160000
D = 256
HALF = D // 2

NC = 2
NS = 16
ET = E // NS
CB = 128
ETP = 10240
CK = ETP // CB
NP = 10240
NR = NP // NS

@functools.cache
def _get_segsum_call():
    mesh = plsc.VectorSubcoreMesh(core_axis_name="c", subcore_axis_name="s",
                                  num_cores=NC, num_subcores=NS)
    return pl.kernel(
        _segsum_body,
        out_type=jax.ShapeDtypeStruct((NC, NP, HALF), jnp.float32),
        mesh=mesh,
        scratch_types=[
            pltpu.VMEM((ETP,), jnp.int32),
            pltpu.VMEM((2, CB), jnp.int32),
            pltpu.VMEM((2, CB, HALF), jnp.float32),
            pltpu.VMEM_SHARED((NP, HALF), jnp.float32),
            pltpu.SemaphoreType.DMA((2,)),
            pltpu.SemaphoreType.DMA((2,)),
        ],
    )


def _segsum_body(x2_hbm, idx2_hbm, dst_hbm, out_hbm,
                 idx_v, dst_v, rows_v, acc, sems, dsems):
    c = lax.axis_index("c")
    s = lax.axis_index("s")

    pltpu.sync_copy(idx2_hbm.at[c, s], idx_v)

    def _zero_row(i, carry):
        for j in range(HALF // 16):
            rows_v[0, i, 16 * j:16 * (j + 1)] = jnp.zeros((16,), jnp.float32)
        return carry
    lax.fori_loop(0, CB, _zero_row, 0)
    for r in range(NR // CB):
        pltpu.sync_copy(rows_v.at[0], acc.at[pl.ds(s * NR + r * CB, CB)])
    plsc.subcore_barrier()

    pltpu.async_copy(dst_hbm.at[s, 0], dst_v.at[0], dsems.at[0])
    pltpu.async_copy(x2_hbm.at[idx_v.at[pl.ds(0, CB)]], rows_v.at[0], sems.at[0])

    def _chunk(k, carry):
        b = k % 2

        @pl.when(k + 1 < CK)
        def _():
            pltpu.async_copy(dst_hbm.at[s, k + 1], dst_v.at[1 - b],
                             dsems.at[1 - b])
            pltpu.async_copy(x2_hbm.at[idx_v.at[pl.ds((k + 1) * CB, CB)]],
                             rows_v.at[1 - b], sems.at[1 - b])

        pltpu.make_async_copy(dst_hbm.at[s, k], dst_v.at[b], dsems.at[b]).wait()
        pltpu.make_async_copy(x2_hbm.at[idx_v.at[pl.ds(k * CB, CB)]],
                              rows_v.at[b], sems.at[b]).wait()
        pltpu.sync_copy(rows_v.at[b], acc.at[dst_v.at[b]], add=True)
        return carry

    lax.fori_loop(0, CK, _chunk, 0)
    plsc.subcore_barrier()

    pltpu.sync_copy(acc.at[pl.ds(s * NR, NR)], out_hbm.at[c, pl.ds(s * NR, NR)])


def _mlp_body(add_residual, fuse_head, x_ref, agg_ref, Wa_ref, ba_ref,
              Wb_ref, bb_ref, g_ref, be_ref, *rest):
    if fuse_head:
        Wh1_ref, bh1_ref, Wh2_ref, bh2_ref, out_ref = rest
    else:
        (out_ref,) = rest
    xb = x_ref[...]
    h0 = xb + jnp.concatenate([agg_ref[0], agg_ref[1]], axis=-1)
    h = jnp.dot(h0, Wa_ref[...], preferred_element_type=jnp.float32)
    h = jnp.maximum(h + ba_ref[...], 0.0)
    h = jnp.dot(h, Wb_ref[...], preferred_element_type=jnp.float32) + bb_ref[...]
    mu = jnp.mean(h, axis=-1, keepdims=True)
    var = jnp.mean((h - mu) ** 2, axis=-1, keepdims=True)
    h = (h - mu) / jnp.sqrt(var + 1e-5) * g_ref[...] + be_ref[...]
    h = jnp.maximum(h, 0.0)
    if add_residual:
        h = h + xb
    if fuse_head:
        h = jnp.maximum(
            jnp.dot(h, Wh1_ref[...], preferred_element_type=jnp.float32)
            + bh1_ref[...], 0.0)
        h = jnp.dot(h, Wh2_ref[...], preferred_element_type=jnp.float32) \
            + bh2_ref[...]
    out_ref[...] = h


_TILE = 1000


def _mlp_call(x, agg, weights, add_residual, fuse_head):
    w_specs = []
    for w in weights:
        if w.ndim == 1:
            w = w.reshape(1, -1)
        w_specs.append((w, pl.BlockSpec(w.shape, lambda i: (0, 0))))
    return pl.pallas_call(
        functools.partial(_mlp_body, add_residual, fuse_head),
        grid=(N // _TILE,),
        in_specs=[
            pl.BlockSpec((_TILE, D), lambda i: (i, 0)),
            pl.BlockSpec((NC, _TILE, HALF), lambda i: (0, i, 0)),
        ] + [spec for _, spec in w_specs],
        out_specs=pl.BlockSpec((_TILE, D), lambda i: (i, 0)),
        out_shape=jax.ShapeDtypeStruct((N, D), jnp.float32),
    )(x, agg, *[w for w, _ in w_specs])


def kernel(x, edge_index, W0a, b0a, W0b, b0b, g0, be0, W1a, b1a, W1b, b1b,
           g1, be1, W2a, b2a, W2b, b2b, g2, be2, Wh1, bh1, Wh2, bh2):
    src = edge_index[0]
    dst = edge_index[1]
    idx2 = jnp.pad(jnp.stack([src * 2, src * 2 + 1]).reshape(NC, NS, ET),
                   ((0, 0), (0, 0), (0, ETP - ET)))
    pad_dst = jnp.broadcast_to(N + jnp.arange(ETP - ET, dtype=jnp.int32),
                               (NS, ETP - ET))
    dstr = jnp.concatenate([dst.reshape(NS, ET), pad_dst],
                           axis=1).reshape(NS, CK, CB)

    def seg(h):
        return _get_segsum_call()(h.reshape(2 * N, HALF), idx2, dstr)

    h = _mlp_call(x, seg(x), (W0a, b0a, W0b, b0b, g0, be0), False, False)
    h = _mlp_call(h, seg(h), (W1a, b1a, W1b, b1b, g1, be1), True, False)
    out = _mlp_call(h, seg(h),
                    (W2a, b2a, W2b, b2b, g2, be2, Wh1, bh1, Wh2, bh2),
                    True, True)
    return out


# speedup vs baseline: 1.7806x; 1.7806x over previous
import functools

import jax
import jax.numpy as jnp
from jax import lax
from jax.experimental import pallas as pl
from jax.experimental.pallas import tpu as pltpu
from jax.experimental.pallas import tpu_sc as plsc

N = 10000
E = 160000
D = 256
HALF = D // 2

NC = 2
NS = 16
ET = E // NS
CB = 96
ETP = 10080
CK = ETP // CB
NP = 10112
NR = NP // NS

@functools.cache
def _get_segsum_call():
    mesh = plsc.VectorSubcoreMesh(core_axis_name="c", subcore_axis_name="s",
                                  num_cores=NC, num_subcores=NS)
    return pl.kernel(
        _segsum_body,
        out_type=jax.ShapeDtypeStruct((NC, NP, HALF), jnp.float32),
        mesh=mesh,
        scratch_types=[
            pltpu.VMEM((ETP,), jnp.int32),
            pltpu.VMEM((CK, CB), jnp.int32),
            pltpu.VMEM((2, CB, HALF), jnp.float32),
            pltpu.VMEM_SHARED((NP, HALF), jnp.float32),
            pltpu.SemaphoreType.DMA((2,)),
        ],
    )


def _segsum_body(x2_hbm, idx2_hbm, dst_hbm, out_hbm,
                 idx_v, dst_v, rows_v, acc, sems):
    c = lax.axis_index("c")
    s = lax.axis_index("s")

    pltpu.sync_copy(idx2_hbm.at[c, s], idx_v)
    pltpu.sync_copy(dst_hbm.at[s], dst_v)

    def _zero_row(i, carry):
        for j in range(HALF // 16):
            rows_v[0, i, 16 * j:16 * (j + 1)] = jnp.zeros((16,), jnp.float32)
        return carry
    lax.fori_loop(0, CB, _zero_row, 0)
    for off in range(0, NR, CB):
        n = min(CB, NR - off)
        pltpu.sync_copy(rows_v.at[0, pl.ds(0, n)],
                        acc.at[pl.ds(s * NR + off, n)])
    plsc.subcore_barrier()

    pltpu.async_copy(x2_hbm.at[idx_v.at[pl.ds(0, CB)]], rows_v.at[0], sems.at[0])

    def _chunk(k, carry):
        b = k % 2

        @pl.when(k + 1 < CK)
        def _():
            pltpu.async_copy(x2_hbm.at[idx_v.at[pl.ds((k + 1) * CB, CB)]],
                             rows_v.at[1 - b], sems.at[1 - b])

        pltpu.make_async_copy(x2_hbm.at[idx_v.at[pl.ds(k * CB, CB)]],
                              rows_v.at[b], sems.at[b]).wait()
        pltpu.sync_copy(rows_v.at[b], acc.at[dst_v.at[k]], add=True)
        return carry

    lax.fori_loop(0, CK, _chunk, 0)
    plsc.subcore_barrier()

    pltpu.sync_copy(acc.at[pl.ds(s * NR, NR)], out_hbm.at[c, pl.ds(s * NR, NR)])


def _mlp_body(add_residual, fuse_head, x_ref, agg_ref, Wa_ref, ba_ref,
              Wb_ref, bb_ref, g_ref, be_ref, *rest):
    if fuse_head:
        Wh1_ref, bh1_ref, Wh2_ref, bh2_ref, out_ref = rest
    else:
        (out_ref,) = rest
    xb = x_ref[...]
    h0 = xb + jnp.concatenate([agg_ref[0], agg_ref[1]], axis=-1)
    h = jnp.dot(h0, Wa_ref[...], preferred_element_type=jnp.float32)
    h = jnp.maximum(h + ba_ref[...], 0.0)
    h = jnp.dot(h, Wb_ref[...], preferred_element_type=jnp.float32) + bb_ref[...]
    mu = jnp.mean(h, axis=-1, keepdims=True)
    var = jnp.mean((h - mu) ** 2, axis=-1, keepdims=True)
    h = (h - mu) / jnp.sqrt(var + 1e-5) * g_ref[...] + be_ref[...]
    h = jnp.maximum(h, 0.0)
    if add_residual:
        h = h + xb
    if fuse_head:
        h = jnp.maximum(
            jnp.dot(h, Wh1_ref[...], preferred_element_type=jnp.float32)
            + bh1_ref[...], 0.0)
        h = jnp.dot(h, Wh2_ref[...], preferred_element_type=jnp.float32) \
            + bh2_ref[...]
    out_ref[...] = h


_TILE = 1000


def _mlp_call(x, agg, weights, add_residual, fuse_head):
    w_specs = []
    for w in weights:
        if w.ndim == 1:
            w = w.reshape(1, -1)
        w_specs.append((w, pl.BlockSpec(w.shape, lambda i: (0, 0))))
    return pl.pallas_call(
        functools.partial(_mlp_body, add_residual, fuse_head),
        grid=(N // _TILE,),
        in_specs=[
            pl.BlockSpec((_TILE, D), lambda i: (i, 0)),
            pl.BlockSpec((NC, _TILE, HALF), lambda i: (0, i, 0)),
        ] + [spec for _, spec in w_specs],
        out_specs=pl.BlockSpec((_TILE, D), lambda i: (i, 0)),
        out_shape=jax.ShapeDtypeStruct((N, D), jnp.float32),
    )(x, agg, *[w for w, _ in w_specs])


def kernel(x, edge_index, W0a, b0a, W0b, b0b, g0, be0, W1a, b1a, W1b, b1b,
           g1, be1, W2a, b2a, W2b, b2b, g2, be2, Wh1, bh1, Wh2, bh2):
    src = edge_index[0]
    dst = edge_index[1]
    idx2 = jnp.pad(jnp.stack([src * 2, src * 2 + 1]).reshape(NC, NS, ET),
                   ((0, 0), (0, 0), (0, ETP - ET)))
    pad_dst = jnp.broadcast_to(N + jnp.arange(ETP - ET, dtype=jnp.int32),
                               (NS, ETP - ET))
    dstr = jnp.concatenate([dst.reshape(NS, ET), pad_dst],
                           axis=1).reshape(NS, CK, CB)

    def seg(h):
        return _get_segsum_call()(h.reshape(2 * N, HALF), idx2, dstr)

    h = _mlp_call(x, seg(x), (W0a, b0a, W0b, b0b, g0, be0), False, False)
    h = _mlp_call(h, seg(h), (W1a, b1a, W1b, b1b, g1, be1), True, False)
    out = _mlp_call(h, seg(h),
                    (W2a, b2a, W2b, b2b, g2, be2, Wh1, bh1, Wh2, bh2),
                    True, True)
    return out


# speedup vs baseline: 2.7816x; 1.5621x over previous
import functools

import jax
import jax.numpy as jnp
from jax import lax
from jax.experimental import pallas as pl
from jax.experimental.pallas import tpu as pltpu
from jax.experimental.pallas import tpu_sc as plsc

N = 10000
E = 160000
D = 256
HALF = D // 2

NC = 2
NS = 16
ET = E // NS
CB = 80
ETP = 10000
CK = ETP // CB
NP = 10240
NR = NP // NS

@functools.cache
def _get_segsum_call():
    mesh = plsc.VectorSubcoreMesh(core_axis_name="c", subcore_axis_name="s",
                                  num_cores=NC, num_subcores=NS)
    return pl.kernel(
        _segsum_body,
        out_type=jax.ShapeDtypeStruct((NC, NP, HALF), jnp.float32),
        mesh=mesh,
        scratch_types=[
            pltpu.VMEM((ETP,), jnp.int32),
            pltpu.VMEM((CK, CB), jnp.int32),
            pltpu.VMEM((2, CB, HALF), jnp.float32),
            pltpu.VMEM_SHARED((NP, HALF), jnp.float32),
            pltpu.SemaphoreType.DMA((2,)),
        ],
    )


def _segsum_body(x2_hbm, idx2_hbm, dst_hbm, out_hbm,
                 idx_v, dst_v, rows_v, acc, sems):
    c = lax.axis_index("c")
    s = lax.axis_index("s")

    pltpu.sync_copy(idx2_hbm.at[c, s], idx_v)
    pltpu.sync_copy(dst_hbm.at[s], dst_v)

    def _zero_row(i, carry):
        for j in range(HALF // 16):
            rows_v[0, i, 16 * j:16 * (j + 1)] = jnp.zeros((16,), jnp.float32)
        return carry
    lax.fori_loop(0, CB, _zero_row, 0)
    for off in range(0, NR, CB):
        n = min(CB, NR - off)
        pltpu.sync_copy(rows_v.at[0, pl.ds(0, n)],
                        acc.at[pl.ds(s * NR + off, n)])
    plsc.subcore_barrier()

    pltpu.async_copy(x2_hbm.at[idx_v.at[pl.ds(0, CB)]], rows_v.at[0], sems.at[0])

    def _chunk(k, carry):
        b = k % 2

        @pl.when(k + 1 < CK)
        def _():
            pltpu.async_copy(x2_hbm.at[idx_v.at[pl.ds((k + 1) * CB, CB)]],
                             rows_v.at[1 - b], sems.at[1 - b])

        pltpu.make_async_copy(x2_hbm.at[idx_v.at[pl.ds(k * CB, CB)]],
                              rows_v.at[b], sems.at[b]).wait()
        pltpu.sync_copy(rows_v.at[b], acc.at[dst_v.at[k]], add=True)
        return carry

    lax.fori_loop(0, CK, _chunk, 0)
    plsc.subcore_barrier()

    pltpu.sync_copy(acc.at[pl.ds(s * NR, NR)], out_hbm.at[c, pl.ds(s * NR, NR)])


def _mlp_body(add_residual, fuse_head, x_ref, agg_ref, Wa_ref, ba_ref,
              Wb_ref, bb_ref, g_ref, be_ref, *rest):
    if fuse_head:
        Wh1_ref, bh1_ref, Wh2_ref, bh2_ref, out_ref = rest
    else:
        (out_ref,) = rest
    xb = x_ref[...]
    h0 = xb + jnp.concatenate([agg_ref[0], agg_ref[1]], axis=-1)
    h = jnp.dot(h0, Wa_ref[...], preferred_element_type=jnp.float32)
    h = jnp.maximum(h + ba_ref[...], 0.0)
    h = jnp.dot(h, Wb_ref[...], preferred_element_type=jnp.float32) + bb_ref[...]
    mu = jnp.mean(h, axis=-1, keepdims=True)
    var = jnp.mean((h - mu) ** 2, axis=-1, keepdims=True)
    h = (h - mu) / jnp.sqrt(var + 1e-5) * g_ref[...] + be_ref[...]
    h = jnp.maximum(h, 0.0)
    if add_residual:
        h = h + xb
    if fuse_head:
        h = jnp.maximum(
            jnp.dot(h, Wh1_ref[...], preferred_element_type=jnp.float32)
            + bh1_ref[...], 0.0)
        h = jnp.dot(h, Wh2_ref[...], preferred_element_type=jnp.float32) \
            + bh2_ref[...]
    out_ref[...] = h


_TILE = 1000


def _mlp_call(x, agg, weights, add_residual, fuse_head):
    w_specs = []
    for w in weights:
        if w.ndim == 1:
            w = w.reshape(1, -1)
        w_specs.append((w, pl.BlockSpec(w.shape, lambda i: (0, 0))))
    return pl.pallas_call(
        functools.partial(_mlp_body, add_residual, fuse_head),
        grid=(N // _TILE,),
        in_specs=[
            pl.BlockSpec((_TILE, D), lambda i: (i, 0)),
            pl.BlockSpec((NC, _TILE, HALF), lambda i: (0, i, 0)),
        ] + [spec for _, spec in w_specs],
        out_specs=pl.BlockSpec((_TILE, D), lambda i: (i, 0)),
        out_shape=jax.ShapeDtypeStruct((N, D), jnp.float32),
    )(x, agg, *[w for w, _ in w_specs])


def kernel(x, edge_index, W0a, b0a, W0b, b0b, g0, be0, W1a, b1a, W1b, b1b,
           g1, be1, W2a, b2a, W2b, b2b, g2, be2, Wh1, bh1, Wh2, bh2):
    src = edge_index[0]
    dst = edge_index[1]
    idx2 = jnp.pad(jnp.stack([src * 2, src * 2 + 1]).reshape(NC, NS, ET),
                   ((0, 0), (0, 0), (0, ETP - ET)))
    pad_dst = jnp.broadcast_to(N + jnp.arange(ETP - ET, dtype=jnp.int32),
                               (NS, ETP - ET))
    dstr = jnp.concatenate([dst.reshape(NS, ET), pad_dst],
                           axis=1).reshape(NS, CK, CB)

    def seg(h):
        return _get_segsum_call()(h.reshape(2 * N, HALF), idx2, dstr)

    h = _mlp_call(x, seg(x), (W0a, b0a, W0b, b0b, g0, be0), False, False)
    h = _mlp_call(h, seg(h), (W1a, b1a, W1b, b1b, g1, be1), True, False)
    out = _mlp_call(h, seg(h),
                    (W2a, b2a, W2b, b2b, g2, be2, Wh1, bh1, Wh2, bh2),
                    True, True)
    return out
